# baseline (speedup 1.0000x reference)
import functools

import jax
import jax.numpy as jnp
from jax.experimental import pallas as pl
import jax.experimental.pallas.tpu as pltpu

TOKENS = 8192
D_IN = 2048
D_OUT = 2048
BM = 512


def _matmul_kernel(x_ref, w_ref, o_ref, w_bf16_ref):
    @pl.when(pl.program_id(0) == 0)
    def _():
        w_bf16_ref[...] = w_ref[...].astype(jnp.bfloat16)

    x_bf16 = x_ref[...].astype(jnp.bfloat16)
    o_ref[...] = jax.lax.dot_general(
        x_bf16,
        w_bf16_ref[...],
        dimension_numbers=(((1,), (1,)), ((), ())),
        preferred_element_type=jnp.float32,
    )


@functools.partial(jax.jit, static_argnames=())
def kernel(x, W):
    out = pl.pallas_call(
        _matmul_kernel,
        grid=(TOKENS // BM,),
        in_specs=[
            pl.BlockSpec((BM, D_IN), lambda i: (i, 0)),
            pl.BlockSpec((D_OUT, D_IN), lambda i: (0, 0)),
        ],
        out_specs=pl.BlockSpec((BM, D_OUT), lambda i: (i, 0)),
        out_shape=jax.ShapeDtypeStruct((TOKENS, D_OUT), jnp.float32),
        scratch_shapes=[pltpu.VMEM((D_OUT, D_IN), jnp.bfloat16)],
    )(x, W)
    return out


# speedup vs baseline: 1.0117x; 1.0117x over previous
import functools

import jax
import jax.numpy as jnp
from jax.experimental import pallas as pl
import jax.experimental.pallas.tpu as pltpu

TOKENS = 8192
D_IN = 2048
D_OUT = 2048
BM = 1024


def _matmul_kernel(x_ref, w_ref, o_ref, w_bf16_ref):
    @pl.when(pl.program_id(0) == 0)
    def _():
        w_bf16_ref[...] = w_ref[...].astype(jnp.bfloat16)

    x_bf16 = x_ref[...].astype(jnp.bfloat16)
    o_ref[...] = jax.lax.dot_general(
        x_bf16,
        w_bf16_ref[...],
        dimension_numbers=(((1,), (1,)), ((), ())),
        preferred_element_type=jnp.float32,
    )


@functools.partial(jax.jit, static_argnames=())
def kernel(x, W):
    out = pl.pallas_call(
        _matmul_kernel,
        grid=(TOKENS // BM,),
        in_specs=[
            pl.BlockSpec((BM, D_IN), lambda i: (i, 0)),
            pl.BlockSpec((D_OUT, D_IN), lambda i: (0, 0)),
        ],
        out_specs=pl.BlockSpec((BM, D_OUT), lambda i: (i, 0)),
        out_shape=jax.ShapeDtypeStruct((TOKENS, D_OUT), jnp.float32),
        scratch_shapes=[pltpu.VMEM((D_OUT, D_IN), jnp.bfloat16)],
        compiler_params=pltpu.CompilerParams(
            vmem_limit_bytes=62 * 1024 * 1024,
        ),
    )(x, W)
    return out
